# baseline (speedup 1.0000x reference)
import jax
import jax.numpy as jnp
from jax.experimental import pallas as pl
from jax.experimental.pallas import tpu as pltpu


_N, _L, _D, _H, _E, _M, _P = 4, 512, 768, 12, 64, 3, 1024


def _hrt_kernel(pos_ref, hidx_ref, tidx_ref, seq_ref, att_ref,
                hs_ref, ts_ref, rs_ref, acc_ref, eemb_ref):
    hh = pl.program_id(1)
    pos = pos_ref[0, 0, :]
    hidx = hidx_ref[0, 0, :]
    tidx = tidx_ref[0, 0, :]

    l_iota = jax.lax.broadcasted_iota(jnp.int32, (_E * _M, _L), 1)
    poh = (pos[:, None] == l_iota).astype(jnp.float32)
    w16 = (poh.reshape(_E, _M, _L).sum(axis=1) * (1.0 / _M)).astype(jnp.bfloat16)

    e_iota = jax.lax.broadcasted_iota(jnp.int32, (_P, _E), 1)
    oh_h = (hidx[:, None] == e_iota).astype(jnp.float32)
    oh_t = (tidx[:, None] == e_iota).astype(jnp.float32)

    @pl.when(hh == 0)
    def _first():
        seq = seq_ref[0]
        mention = jnp.dot(poh, seq, preferred_element_type=jnp.float32)
        me = mention.reshape(_E, _M, _D)
        mmax = jnp.max(me, axis=1)
        e_emb = mmax + jnp.log(jnp.sum(jnp.exp(me - mmax[:, None, :]), axis=1))
        eemb_ref[...] = e_emb
        hs_ref[0] = jnp.dot(oh_h, e_emb, preferred_element_type=jnp.float32)
        ts_ref[0] = jnp.dot(oh_t, e_emb, preferred_element_type=jnp.float32)
        acc_ref[...] = jnp.zeros((_P, _L), jnp.float32)

    att_h = att_ref[0, 0].astype(jnp.bfloat16)
    e_att_h = jnp.dot(w16, att_h,
                      preferred_element_type=jnp.float32).astype(jnp.bfloat16)
    h_att = jnp.dot(oh_h.astype(jnp.bfloat16), e_att_h,
                    preferred_element_type=jnp.float32)
    t_att = jnp.dot(oh_t.astype(jnp.bfloat16), e_att_h,
                    preferred_element_type=jnp.float32)
    acc_ref[...] += h_att * t_att

    @pl.when(hh == _H - 1)
    def _last():
        ht_att = acc_ref[...] * (1.0 / _H)
        ht_att = ht_att / (jnp.sum(ht_att, axis=1, keepdims=True) + 1e-5)
        rs_ref[0] = jnp.dot(ht_att.astype(jnp.bfloat16),
                            seq_ref[0].astype(jnp.bfloat16),
                            preferred_element_type=jnp.float32)


def kernel(sequence_output, attention, entity_pos, hts):
    n, L, d = sequence_output.shape
    h = attention.shape[1]
    E, M = entity_pos.shape[1], entity_pos.shape[2]
    P = hts.shape[1]
    assert (n, L, d, h, E, M, P) == (_N, _L, _D, _H, _E, _M, _P)

    pos = (entity_pos[:, :, :, 0].reshape(n, 1, E * M) + 1).astype(jnp.int32)
    hidx = hts[:, :, 0].reshape(n, 1, P).astype(jnp.int32)
    tidx = hts[:, :, 1].reshape(n, 1, P).astype(jnp.int32)

    out_shape = [jax.ShapeDtypeStruct((n, P, d), jnp.float32)] * 3
    hs, ts, rs = pl.pallas_call(
        _hrt_kernel,
        grid=(n, h),
        in_specs=[
            pl.BlockSpec((1, 1, E * M), lambda i, j: (i, 0, 0)),
            pl.BlockSpec((1, 1, P), lambda i, j: (i, 0, 0)),
            pl.BlockSpec((1, 1, P), lambda i, j: (i, 0, 0)),
            pl.BlockSpec((1, L, d), lambda i, j: (i, 0, 0)),
            pl.BlockSpec((1, 1, L, L), lambda i, j: (i, j, 0, 0)),
        ],
        out_specs=[
            pl.BlockSpec((1, P, d), lambda i, j: (i, 0, 0)),
            pl.BlockSpec((1, P, d), lambda i, j: (i, 0, 0)),
            pl.BlockSpec((1, P, d), lambda i, j: (i, 0, 0)),
        ],
        out_shape=out_shape,
        scratch_shapes=[
            pltpu.VMEM((_P, _L), jnp.float32),
            pltpu.VMEM((_E, _D), jnp.float32),
        ],
    )(pos, hidx, tidx, sequence_output, attention)

    return hs.reshape(-1, d), ts.reshape(-1, d), rs.reshape(-1, d)


# speedup vs baseline: 1.7550x; 1.7550x over previous
import jax
import jax.numpy as jnp
from jax.experimental import pallas as pl


_N, _L, _D, _H, _E, _M, _P = 4, 512, 768, 12, 64, 3, 1024


def _hrt_kernel(pos_ref, hidx_ref, tidx_ref, seq_ref, att_ref,
                hs_ref, ts_ref, rs_ref):
    seq = seq_ref[0]
    seq16 = seq.astype(jnp.bfloat16)
    pos = pos_ref[0, 0, :]
    hidx = hidx_ref[0, 0, :]
    tidx = tidx_ref[0, 0, :]

    l_iota = jax.lax.broadcasted_iota(jnp.int32, (_E * _M, _L), 1)
    poh = (pos[:, None] == l_iota).astype(jnp.bfloat16)

    mention = jnp.dot(poh, seq16, preferred_element_type=jnp.float32)
    me = mention.reshape(_E, _M, _D)
    mmax = jnp.max(me, axis=1)
    e_emb = mmax + jnp.log(jnp.sum(jnp.exp(me - mmax[:, None, :]), axis=1))

    w16 = (poh.reshape(_E, _M, _L).sum(axis=1) * (1.0 / _M))

    e_iota = jax.lax.broadcasted_iota(jnp.int32, (_P, _E), 1)
    oh_h = (hidx[:, None] == e_iota).astype(jnp.bfloat16)
    oh_t = (tidx[:, None] == e_iota).astype(jnp.bfloat16)

    acc = jnp.zeros((_P, _L), jnp.float32)
    for hh in range(_H):
        att_h = att_ref[0, hh].astype(jnp.bfloat16)
        e_att_h = jnp.dot(w16, att_h,
                          preferred_element_type=jnp.float32
                          ).astype(jnp.bfloat16)
        h_att = jnp.dot(oh_h, e_att_h, preferred_element_type=jnp.float32)
        t_att = jnp.dot(oh_t, e_att_h, preferred_element_type=jnp.float32)
        acc = acc + h_att * t_att

    ht_att = acc * (1.0 / _H)
    ht_att = ht_att / (jnp.sum(ht_att, axis=1, keepdims=True) + 1e-5)

    rs_ref[0] = jnp.dot(ht_att.astype(jnp.bfloat16), seq16,
                        preferred_element_type=jnp.float32)
    e_emb16 = e_emb.astype(jnp.bfloat16)
    hs_ref[0] = jnp.dot(oh_h, e_emb16, preferred_element_type=jnp.float32)
    ts_ref[0] = jnp.dot(oh_t, e_emb16, preferred_element_type=jnp.float32)


def kernel(sequence_output, attention, entity_pos, hts):
    n, L, d = sequence_output.shape
    h = attention.shape[1]
    E, M = entity_pos.shape[1], entity_pos.shape[2]
    P = hts.shape[1]
    assert (n, L, d, h, E, M, P) == (_N, _L, _D, _H, _E, _M, _P)

    pos = (entity_pos[:, :, :, 0].reshape(n, 1, E * M) + 1).astype(jnp.int32)
    hidx = hts[:, :, 0].reshape(n, 1, P).astype(jnp.int32)
    tidx = hts[:, :, 1].reshape(n, 1, P).astype(jnp.int32)

    out_shape = [jax.ShapeDtypeStruct((n, P, d), jnp.float32)] * 3
    hs, ts, rs = pl.pallas_call(
        _hrt_kernel,
        grid=(n,),
        in_specs=[
            pl.BlockSpec((1, 1, E * M), lambda i: (i, 0, 0)),
            pl.BlockSpec((1, 1, P), lambda i: (i, 0, 0)),
            pl.BlockSpec((1, 1, P), lambda i: (i, 0, 0)),
            pl.BlockSpec((1, L, d), lambda i: (i, 0, 0)),
            pl.BlockSpec((1, h, L, L), lambda i: (i, 0, 0, 0)),
        ],
        out_specs=[
            pl.BlockSpec((1, P, d), lambda i: (i, 0, 0)),
            pl.BlockSpec((1, P, d), lambda i: (i, 0, 0)),
            pl.BlockSpec((1, P, d), lambda i: (i, 0, 0)),
        ],
        out_shape=out_shape,
    )(pos, hidx, tidx, sequence_output, attention)

    return hs.reshape(-1, d), ts.reshape(-1, d), rs.reshape(-1, d)


# speedup vs baseline: 1.8573x; 1.0583x over previous
import jax
import jax.numpy as jnp
from jax.experimental import pallas as pl


_N, _L, _D, _H, _E, _M, _P = 4, 512, 768, 12, 64, 3, 1024


def _hrt_kernel(pos_ref, hidx_ref, tidx_ref, seq_ref, att_ref,
                hs_ref, ts_ref, rs_ref):
    seq = seq_ref[0]
    seq16 = seq.astype(jnp.bfloat16)
    pos = pos_ref[0, 0, :]
    hidx = hidx_ref[0, 0, :]
    tidx = tidx_ref[0, 0, :]

    l_iota = jax.lax.broadcasted_iota(jnp.int32, (_E * _M, _L), 1)
    poh = (pos[:, None] == l_iota).astype(jnp.bfloat16)

    mention = jnp.dot(poh, seq16, preferred_element_type=jnp.float32)
    me = mention.reshape(_E, _M, _D)
    mmax = jnp.max(me, axis=1)
    e_emb = mmax + jnp.log(jnp.sum(jnp.exp(me - mmax[:, None, :]), axis=1))

    w16 = (poh.reshape(_E, _M, _L).sum(axis=1) * (1.0 / _M))

    e_iota = jax.lax.broadcasted_iota(jnp.int32, (_P, _E), 1)
    oh_h = (hidx[:, None] == e_iota).astype(jnp.bfloat16)
    oh_t = (tidx[:, None] == e_iota).astype(jnp.bfloat16)

    e_att_cols = []
    for hh in range(_H):
        att_h = att_ref[0, hh].astype(jnp.bfloat16)
        e_att_cols.append(jnp.dot(w16, att_h,
                                  preferred_element_type=jnp.float32
                                  ).astype(jnp.bfloat16))
    e_att_all = jnp.concatenate(e_att_cols, axis=1)

    hc = 4
    acc = jnp.zeros((_P, _L), jnp.float32)
    for c in range(_H // hc):
        ec = e_att_all[:, c * hc * _L:(c + 1) * hc * _L]
        h_att = jnp.dot(oh_h, ec, preferred_element_type=jnp.float32)
        t_att = jnp.dot(oh_t, ec, preferred_element_type=jnp.float32)
        prod = h_att * t_att
        for k in range(hc):
            acc = acc + prod[:, k * _L:(k + 1) * _L]

    ht_att = acc * (1.0 / _H)
    ht_att = ht_att / (jnp.sum(ht_att, axis=1, keepdims=True) + 1e-5)

    rs_ref[0] = jnp.dot(ht_att.astype(jnp.bfloat16), seq16,
                        preferred_element_type=jnp.float32)
    e_emb16 = e_emb.astype(jnp.bfloat16)
    hs_ref[0] = jnp.dot(oh_h, e_emb16, preferred_element_type=jnp.float32)
    ts_ref[0] = jnp.dot(oh_t, e_emb16, preferred_element_type=jnp.float32)


def kernel(sequence_output, attention, entity_pos, hts):
    n, L, d = sequence_output.shape
    h = attention.shape[1]
    E, M = entity_pos.shape[1], entity_pos.shape[2]
    P = hts.shape[1]
    assert (n, L, d, h, E, M, P) == (_N, _L, _D, _H, _E, _M, _P)

    pos = (entity_pos[:, :, :, 0].reshape(n, 1, E * M) + 1).astype(jnp.int32)
    hidx = hts[:, :, 0].reshape(n, 1, P).astype(jnp.int32)
    tidx = hts[:, :, 1].reshape(n, 1, P).astype(jnp.int32)

    out_shape = [jax.ShapeDtypeStruct((n, P, d), jnp.float32)] * 3
    hs, ts, rs = pl.pallas_call(
        _hrt_kernel,
        grid=(n,),
        in_specs=[
            pl.BlockSpec((1, 1, E * M), lambda i: (i, 0, 0)),
            pl.BlockSpec((1, 1, P), lambda i: (i, 0, 0)),
            pl.BlockSpec((1, 1, P), lambda i: (i, 0, 0)),
            pl.BlockSpec((1, L, d), lambda i: (i, 0, 0)),
            pl.BlockSpec((1, h, L, L), lambda i: (i, 0, 0, 0)),
        ],
        out_specs=[
            pl.BlockSpec((1, P, d), lambda i: (i, 0, 0)),
            pl.BlockSpec((1, P, d), lambda i: (i, 0, 0)),
            pl.BlockSpec((1, P, d), lambda i: (i, 0, 0)),
        ],
        out_shape=out_shape,
    )(pos, hidx, tidx, sequence_output, attention)

    return hs.reshape(-1, d), ts.reshape(-1, d), rs.reshape(-1, d)
